# Initial kernel scaffold; baseline (speedup 1.0000x reference)
import jax
import jax.numpy as jnp
from jax.experimental import pallas as pl


def kernel(user_emb, item_emb, edge_index):
    raise NotImplementedError("write your pallas kernel here")


# speedup vs baseline: 1.2027x; 1.2027x over previous
import functools

import jax
import jax.numpy as jnp
from jax import lax
from jax.experimental import pallas as pl
from jax.experimental.pallas import tpu as pltpu
from jax.experimental.pallas import tpu_sc as plsc

D = 128
L = 16
NC, NS = 2, 16
NW = NC * NS
CHUNK = 400


def _make_sc_kernel(n_edges):
    assert n_edges % (NW * 8) == 0
    per_w = n_edges // NW
    assert per_w % CHUNK == 0
    n_chunks = per_w // CHUNK
    mesh = plsc.VectorSubcoreMesh(
        core_axis_name="c", subcore_axis_name="s", num_cores=NC, num_subcores=NS
    )

    @functools.partial(
        pl.kernel,
        out_type=jax.ShapeDtypeStruct((n_edges,), jnp.float32),
        mesh=mesh,
        compiler_params=pltpu.CompilerParams(needs_layout_passes=False),
        scratch_types=[
            pltpu.VMEM((CHUNK,), jnp.int32),
            pltpu.VMEM((CHUNK,), jnp.int32),
            pltpu.VMEM((CHUNK, D), jnp.float32),
            pltpu.VMEM((CHUNK, D), jnp.float32),
            pltpu.VMEM((CHUNK,), jnp.float32),
            pltpu.SemaphoreType.DMA,
            pltpu.SemaphoreType.DMA,
        ],
    )
    def sc_kernel(user_hbm, item_hbm, uidx_hbm, iidx_hbm, out_hbm,
                  uidx_v, iidx_v, urows_v, irows_v, out_v, usem, isem):
        wid = lax.axis_index("s") * NC + lax.axis_index("c")
        wbase = wid * per_w
        lane = lax.iota(jnp.int32, L)

        def chunk_body(c, _):
            base = wbase + c * CHUNK
            pltpu.sync_copy(uidx_hbm.at[pl.ds(base, CHUNK)], uidx_v)
            pltpu.sync_copy(iidx_hbm.at[pl.ds(base, CHUNK)], iidx_v)
            cu = pltpu.async_copy(user_hbm.at[uidx_v], urows_v, usem)
            ci = pltpu.async_copy(item_hbm.at[iidx_v], irows_v, isem)
            cu.wait()
            ci.wait()

            def group_body(g, _):
                eidx = g * L + lane
                acc = plsc.load_gather(urows_v, [eidx, lane * 0]) * plsc.load_gather(
                    irows_v, [eidx, lane * 0])
                for d in range(1, D):
                    cd = jnp.full((L,), d, jnp.int32)
                    acc += plsc.load_gather(urows_v, [eidx, cd]) * plsc.load_gather(
                        irows_v, [eidx, cd])
                out_v[pl.ds(g * L, L)] = acc
                return 0

            lax.fori_loop(0, CHUNK // L, group_body, 0)
            pltpu.sync_copy(out_v, out_hbm.at[pl.ds(base, CHUNK)])
            return 0

        lax.fori_loop(0, n_chunks, chunk_body, 0)

    return sc_kernel


@jax.jit
def kernel(user_emb, item_emb, edge_index):
    n_edges = edge_index.shape[1]
    sc = _make_sc_kernel(n_edges)
    return sc(user_emb, item_emb, edge_index[0], edge_index[1])


# speedup vs baseline: 3.4193x; 2.8431x over previous
import functools

import jax
import jax.numpy as jnp
from jax import lax
from jax.experimental import pallas as pl
from jax.experimental.pallas import tpu as pltpu
from jax.experimental.pallas import tpu_sc as plsc

D = 128
DP = 129
L = 16
NC, NS = 2, 16
NW = NC * NS
CHUNK = 400


def _make_sc_kernel(n_edges):
    assert n_edges % (NW * 8) == 0
    per_w = n_edges // NW
    assert per_w % CHUNK == 0
    n_chunks = per_w // CHUNK
    mesh = plsc.VectorSubcoreMesh(
        core_axis_name="c", subcore_axis_name="s", num_cores=NC, num_subcores=NS
    )

    @functools.partial(
        pl.kernel,
        out_type=jax.ShapeDtypeStruct((n_edges,), jnp.float32),
        mesh=mesh,
        compiler_params=pltpu.CompilerParams(
            needs_layout_passes=False, use_tc_tiling_on_sc=False
        ),
        scratch_types=[
            pltpu.VMEM((CHUNK,), jnp.int32),
            pltpu.VMEM((CHUNK,), jnp.int32),
            pltpu.VMEM((CHUNK, DP), jnp.float32),
            pltpu.VMEM((CHUNK, DP), jnp.float32),
            pltpu.VMEM((CHUNK,), jnp.float32),
            pltpu.SemaphoreType.DMA,
            pltpu.SemaphoreType.DMA,
        ],
    )
    def sc_kernel(user_hbm, item_hbm, uidx_hbm, iidx_hbm, out_hbm,
                  uidx_v, iidx_v, urows_v, irows_v, out_v, usem, isem):
        wid = lax.axis_index("s") * NC + lax.axis_index("c")
        wbase = wid * per_w
        lane = lax.iota(jnp.int32, L)

        def chunk_body(c, _):
            base = wbase + c * CHUNK
            pltpu.sync_copy(uidx_hbm.at[pl.ds(base, CHUNK)], uidx_v)
            pltpu.sync_copy(iidx_hbm.at[pl.ds(base, CHUNK)], iidx_v)
            cu = pltpu.async_copy(user_hbm.at[uidx_v], urows_v, usem)
            ci = pltpu.async_copy(item_hbm.at[iidx_v], irows_v, isem)
            cu.wait()
            ci.wait()

            def group_body(g, _):
                eidx = g * L + lane
                acc = plsc.load_gather(urows_v, [eidx, lane * 0]) * plsc.load_gather(
                    irows_v, [eidx, lane * 0])
                for d in range(1, D):
                    cd = jnp.full((L,), d, jnp.int32)
                    acc += plsc.load_gather(urows_v, [eidx, cd]) * plsc.load_gather(
                        irows_v, [eidx, cd])
                out_v[pl.ds(g * L, L)] = acc
                return 0

            lax.fori_loop(0, CHUNK // L, group_body, 0)
            pltpu.sync_copy(out_v, out_hbm.at[pl.ds(base, CHUNK)])
            return 0

        lax.fori_loop(0, n_chunks, chunk_body, 0)

    return sc_kernel


@jax.jit
def kernel(user_emb, item_emb, edge_index):
    n_edges = edge_index.shape[1]
    sc = _make_sc_kernel(n_edges)
    upad = jnp.pad(user_emb, ((0, 0), (0, DP - D)))
    ipad = jnp.pad(item_emb, ((0, 0), (0, DP - D)))
    return sc(upad, ipad, edge_index[0], edge_index[1])


# speedup vs baseline: 4.8416x; 1.4160x over previous
import functools

import jax
import jax.numpy as jnp
from jax import lax
from jax.experimental import pallas as pl
from jax.experimental.pallas import tpu as pltpu
from jax.experimental.pallas import tpu_sc as plsc

D = 128
L = 16
NC, NS = 2, 16
NW = NC * NS
CHUNK = 400


def _make_sc_kernel(n_edges):
    assert n_edges % (NW * 8) == 0
    per_w = n_edges // NW
    assert per_w % CHUNK == 0
    n_chunks = per_w // CHUNK
    mesh = plsc.VectorSubcoreMesh(
        core_axis_name="c", subcore_axis_name="s", num_cores=NC, num_subcores=NS
    )

    @functools.partial(
        pl.kernel,
        out_type=jax.ShapeDtypeStruct((n_edges,), jnp.float32),
        mesh=mesh,
        compiler_params=pltpu.CompilerParams(
            needs_layout_passes=False, use_tc_tiling_on_sc=False
        ),
        scratch_types=[
            pltpu.VMEM((CHUNK,), jnp.int32),
            pltpu.VMEM((CHUNK,), jnp.int32),
            pltpu.VMEM((CHUNK, D), jnp.float32),
            pltpu.VMEM((CHUNK, D), jnp.float32),
            pltpu.VMEM((CHUNK,), jnp.float32),
            pltpu.SemaphoreType.DMA,
            pltpu.SemaphoreType.DMA,
        ],
    )
    def sc_kernel(user_hbm, item_hbm, uidx_hbm, iidx_hbm, out_hbm,
                  uidx_v, iidx_v, urows_v, irows_v, out_v, usem, isem):
        wid = lax.axis_index("s") * NC + lax.axis_index("c")
        wbase = wid * per_w
        lane = lax.iota(jnp.int32, L)

        def chunk_body(c, _):
            base = wbase + c * CHUNK
            pltpu.sync_copy(uidx_hbm.at[pl.ds(base, CHUNK)], uidx_v)
            pltpu.sync_copy(iidx_hbm.at[pl.ds(base, CHUNK)], iidx_v)
            cu = pltpu.async_copy(user_hbm.at[uidx_v], urows_v, usem)
            ci = pltpu.async_copy(item_hbm.at[iidx_v], irows_v, isem)
            cu.wait()
            ci.wait()

            def group_body(g, _):
                eidx = g * L + lane
                col = lane
                acc = plsc.load_gather(urows_v, [eidx, col]) * plsc.load_gather(
                    irows_v, [eidx, col])
                for d in range(1, D):
                    col = (lane + d) & (D - 1)
                    acc += plsc.load_gather(urows_v, [eidx, col]) * plsc.load_gather(
                        irows_v, [eidx, col])
                out_v[pl.ds(g * L, L)] = acc
                return 0

            lax.fori_loop(0, CHUNK // L, group_body, 0)
            pltpu.sync_copy(out_v, out_hbm.at[pl.ds(base, CHUNK)])
            return 0

        lax.fori_loop(0, n_chunks, chunk_body, 0)

    return sc_kernel


@jax.jit
def kernel(user_emb, item_emb, edge_index):
    n_edges = edge_index.shape[1]
    sc = _make_sc_kernel(n_edges)
    return sc(user_emb, item_emb, edge_index[0], edge_index[1])
